# Initial kernel scaffold; baseline (speedup 1.0000x reference)
import jax
import jax.numpy as jnp
from jax.experimental import pallas as pl


def kernel(x, hidden, emb):
    raise NotImplementedError("write your pallas kernel here")


# speedup vs baseline: 1.5295x; 1.5295x over previous
import functools

import jax
import jax.numpy as jnp
from jax import lax
from jax.experimental import pallas as pl
from jax.experimental.pallas import tpu as pltpu
from jax.experimental.pallas import tpu_sc as plsc

SEQ = 200
BATCH = 4096
D = 32
NROWS = SEQ * BATCH
RPC = 128
GPB = 8
CHUNK = GPB * RPC


def _make_kernel():
  info = plsc.get_sparse_core_info()
  nc, ns = info.num_cores, info.num_subcores
  nw = nc * ns
  rows_per_w = NROWS // nw
  irows_per_w = rows_per_w // RPC
  iters = irows_per_w // GPB

  mesh = plsc.VectorSubcoreMesh(core_axis_name="c", subcore_axis_name="s")

  @functools.partial(
      pl.kernel,
      mesh=mesh,
      compiler_params=pltpu.CompilerParams(use_tc_tiling_on_sc=False),
      out_type=jax.ShapeDtypeStruct((NROWS // RPC, RPC, D), jnp.float32),
      scratch_types=[
          pltpu.VMEM((GPB, RPC), jnp.int32),
          pltpu.VMEM((GPB, RPC, D), jnp.float32),
          pltpu.SemaphoreType.DMA,
      ],
  )
  def gather_kernel(emb_hbm, idx_hbm, out_hbm, idx_v, rows_v, sem):
    wid = lax.axis_index("s") * nc + lax.axis_index("c")
    irow0 = wid * irows_per_w

    def body(it, carry):
      base = irow0 + it * GPB
      pltpu.sync_copy(idx_hbm.at[pl.ds(base, GPB)], idx_v)
      copies = [
          pltpu.async_copy(emb_hbm.at[idx_v.at[j]], rows_v.at[j], sem)
          for j in range(GPB)
      ]
      for c in copies:
        c.wait()
      pltpu.sync_copy(rows_v, out_hbm.at[pl.ds(base, GPB)])
      return carry

    lax.fori_loop(0, iters, body, 0)

  return gather_kernel


_gather = _make_kernel()


def kernel(x, hidden, emb):
  del hidden
  idx = jnp.transpose(x.astype(jnp.int32)).reshape(NROWS // RPC, RPC)
  out = _gather(emb, idx)
  return out.reshape(SEQ, BATCH, D)


# speedup vs baseline: 1.5664x; 1.0241x over previous
import functools

import jax
import jax.numpy as jnp
from jax import lax
from jax.experimental import pallas as pl
from jax.experimental.pallas import tpu as pltpu
from jax.experimental.pallas import tpu_sc as plsc

SEQ = 200
BATCH = 4096
D = 32
NROWS = SEQ * BATCH
RPC = 128
GPB = 10
CHUNK = GPB * RPC


def _make_kernel():
  info = plsc.get_sparse_core_info()
  nc, ns = info.num_cores, info.num_subcores
  nw = nc * ns
  rows_per_w = NROWS // nw
  irows_per_w = rows_per_w // RPC
  iters = irows_per_w // GPB
  n2 = iters // 2

  mesh = plsc.VectorSubcoreMesh(core_axis_name="c", subcore_axis_name="s")

  @functools.partial(
      pl.kernel,
      mesh=mesh,
      compiler_params=pltpu.CompilerParams(use_tc_tiling_on_sc=False),
      out_type=jax.ShapeDtypeStruct((NROWS // RPC, RPC, D), jnp.float32),
      scratch_types=[
          pltpu.VMEM((GPB, RPC), jnp.int32),
          pltpu.VMEM((GPB, RPC), jnp.int32),
          pltpu.VMEM((GPB, RPC, D), jnp.float32),
          pltpu.VMEM((GPB, RPC, D), jnp.float32),
          pltpu.SemaphoreType.DMA,
          pltpu.SemaphoreType.DMA,
          pltpu.SemaphoreType.DMA,
          pltpu.SemaphoreType.DMA,
      ],
  )
  def gather_kernel(emb_hbm, idx_hbm, out_hbm, idx_v0, idx_v1,
                    rows_v0, rows_v1, sg0, sg1, so0, so1):
    wid = lax.axis_index("s") * nc + lax.axis_index("c")
    irow0 = wid * irows_per_w

    def fire(idx_v, rows_v, sem, base):
      pltpu.sync_copy(idx_hbm.at[pl.ds(base, GPB)], idx_v)
      for j in range(GPB):
        pltpu.async_copy(emb_hbm.at[idx_v.at[j]], rows_v.at[j], sem)

    def drain_g(idx_v, rows_v, sem):
      for j in range(GPB):
        pltpu.make_async_copy(emb_hbm.at[idx_v.at[j]], rows_v.at[j], sem).wait()

    def out_start(rows_v, sem, base):
      pltpu.async_copy(rows_v, out_hbm.at[pl.ds(base, GPB)], sem)

    def out_drain(rows_v, sem, base):
      pltpu.make_async_copy(rows_v, out_hbm.at[pl.ds(base, GPB)], sem).wait()

    fire(idx_v0, rows_v0, sg0, irow0)

    def body(g2, carry):
      base0 = irow0 + (2 * g2) * GPB
      base1 = base0 + GPB

      @pl.when(g2 > 0)
      def _():
        out_drain(rows_v1, so1, base0 - GPB)
      fire(idx_v1, rows_v1, sg1, base1)
      drain_g(idx_v0, rows_v0, sg0)
      out_start(rows_v0, so0, base0)

      out_drain(rows_v0, so0, base0)
      @pl.when(g2 < n2 - 1)
      def _():
        fire(idx_v0, rows_v0, sg0, base1 + GPB)
      drain_g(idx_v1, rows_v1, sg1)
      out_start(rows_v1, so1, base1)
      return carry

    lax.fori_loop(0, n2, body, 0)

    out_drain(rows_v1, so1, irow0 + (iters - 1) * GPB)

  return gather_kernel


_gather = _make_kernel()


def kernel(x, hidden, emb):
  del hidden
  idx = jnp.transpose(x.astype(jnp.int32)).reshape(NROWS // RPC, RPC)
  out = _gather(emb, idx)
  return out.reshape(SEQ, BATCH, D)


# speedup vs baseline: 1.5745x; 1.0052x over previous
import functools

import jax
import jax.numpy as jnp
from jax import lax
from jax.experimental import pallas as pl
from jax.experimental.pallas import tpu as pltpu
from jax.experimental.pallas import tpu_sc as plsc

SEQ = 200
BATCH = 4096
D = 32
BPW = 128
GPB = 10


def _make_kernel():
  info = plsc.get_sparse_core_info()
  nc, ns = info.num_cores, info.num_subcores
  nw = nc * ns
  assert BPW * nw == BATCH
  iters = SEQ // GPB
  n2 = iters // 2

  mesh = plsc.VectorSubcoreMesh(core_axis_name="c", subcore_axis_name="s")

  @functools.partial(
      pl.kernel,
      mesh=mesh,
      compiler_params=pltpu.CompilerParams(use_tc_tiling_on_sc=False),
      out_type=jax.ShapeDtypeStruct((SEQ, BATCH, D), jnp.float32),
      scratch_types=[
          pltpu.VMEM((SEQ, BPW), jnp.int32),
          pltpu.VMEM((GPB, BPW, D), jnp.float32),
          pltpu.VMEM((GPB, BPW, D), jnp.float32),
          pltpu.SemaphoreType.DMA,
          pltpu.SemaphoreType.DMA,
          pltpu.SemaphoreType.DMA,
          pltpu.SemaphoreType.DMA,
      ],
  )
  def gather_kernel(emb_hbm, xt_hbm, out_hbm, idx_v,
                    rows_v0, rows_v1, sg0, sg1, so0, so1):
    wid = lax.axis_index("s") * nc + lax.axis_index("c")
    b0 = wid * BPW

    pltpu.sync_copy(xt_hbm.at[:, pl.ds(b0, BPW)], idx_v)

    def fire(rows_v, sem, l0):
      for j in range(GPB):
        pltpu.async_copy(emb_hbm.at[idx_v.at[l0 + j]], rows_v.at[j], sem)

    def drain_g(rows_v, sem, l0):
      for j in range(GPB):
        pltpu.make_async_copy(
            emb_hbm.at[idx_v.at[l0 + j]], rows_v.at[j], sem).wait()

    def out_start(rows_v, sem, l0):
      for j in range(GPB):
        pltpu.async_copy(rows_v.at[j], out_hbm.at[l0 + j, pl.ds(b0, BPW)], sem)

    def out_drain(rows_v, sem, l0):
      for j in range(GPB):
        pltpu.make_async_copy(
            rows_v.at[j], out_hbm.at[l0 + j, pl.ds(b0, BPW)], sem).wait()

    fire(rows_v0, sg0, 0)

    def body(g2, carry):
      l0 = (2 * g2) * GPB
      l1 = l0 + GPB

      @pl.when(g2 > 0)
      def _():
        out_drain(rows_v1, so1, l0 - GPB)
      fire(rows_v1, sg1, l1)
      drain_g(rows_v0, sg0, l0)
      out_start(rows_v0, so0, l0)

      out_drain(rows_v0, so0, l0)
      @pl.when(g2 < n2 - 1)
      def _():
        fire(rows_v0, sg0, l1 + GPB)
      drain_g(rows_v1, sg1, l1)
      out_start(rows_v1, so1, l1)
      return carry

    lax.fori_loop(0, n2, body, 0)

    out_drain(rows_v1, so1, (iters - 1) * GPB)

  return gather_kernel


_gather = _make_kernel()


def kernel(x, hidden, emb):
  del hidden
  xt = jnp.transpose(x.astype(jnp.int32))
  return _gather(emb, xt)
